# baseline (speedup 1.0000x reference)
import functools

import jax
import jax.numpy as jnp
from jax import lax
from jax.experimental import pallas as pl
from jax.experimental.pallas import tpu as pltpu
from jax.experimental.pallas import tpu_sc as plsc

N = 10000
NP = 10240
NPD = NP + 8
E = 160000
CK = 128
NCK = E // CK
HALF = NP // 2
AFLAT = NP * NP
APAD = AFLAT + 8
ZR = 256
NPH = NP + 16


def _mesh():
    return plsc.VectorSubcoreMesh(core_axis_name="c", subcore_axis_name="s")


def _fill_zero_1d(ref, n):
    def body(i, _):
        ref[pl.ds(i * 16, 16)] = jnp.zeros((16,), ref.dtype)
        return _
    lax.fori_loop(0, n // 16, body, None)


def _fill_zero_2d(ref, rows, cols):
    def body(i, _):
        r = i // (cols // 16)
        c = (i % (cols // 16)) * 16
        ref[r, pl.ds(c, 16)] = jnp.zeros((16,), ref.dtype)
        return _
    lax.fori_loop(0, rows * (cols // 16), body, None)


def _sc_edges_body(ei, a_out, degp, selfp,
                   src_v, dst_v, aidx_v, ones_v, zer_v,
                   deg_v, self_v, sem):
    c = lax.axis_index("c")
    s = lax.axis_index("s")
    wid = c * 16 + s

    _fill_zero_1d(zer_v, 16384)
    _fill_zero_1d(deg_v, NPH)
    _fill_zero_1d(self_v, NPH)
    def ones_body(i, _):
        ones_v[pl.ds(i * 16, 16)] = jnp.ones((16,), jnp.float32)
        return _
    lax.fori_loop(0, CK // 16, ones_body, None)

    per_tile = HALF * NP // 16
    base = c * (HALF * NP) + s * per_tile
    def zbody(i, _):
        pltpu.sync_copy(zer_v, a_out.at[pl.ds(base + i * 16384, 16384)])
        return _
    lax.fori_loop(0, per_tile // 16384, zbody, None)

    plsc.subcore_barrier()

    nb = NCK // 32
    rem = NCK - nb * 32
    h_start = wid * nb + jnp.minimum(wid, rem)
    h_cnt = nb + (wid < rem).astype(jnp.int32)
    ones16 = jnp.ones((16,), jnp.float32)

    def hist_chunk(t, _):
        ck = h_start + t
        pltpu.sync_copy(ei.at[0, pl.ds(ck * CK, CK)], src_v)
        pltpu.sync_copy(ei.at[1, pl.ds(ck * CK, CK)], dst_v)

        def lane(i, _):
            sv = src_v[pl.ds(i * 16, 16)]
            dv = dst_v[pl.ds(i * 16, 16)]
            didx = jnp.where(sv != dv, dv, NP)
            sidx = jnp.where(sv == dv, sv, NP)
            plsc.addupdate_scatter(deg_v, [didx], ones16)
            plsc.addupdate_scatter(self_v, [sidx], ones16)
            return _
        lax.fori_loop(0, CK // 16, lane, None)
        return _
    lax.fori_loop(0, h_cnt, hist_chunk, None)

    pltpu.sync_copy(deg_v, degp.at[pl.ds(wid * NPH, NPH)])
    pltpu.sync_copy(self_v, selfp.at[pl.ds(wid * NPH, NPH)])

    nb2 = NCK // 16
    rem2 = NCK - nb2 * 16
    a_start = s * nb2 + jnp.minimum(s, rem2)
    a_cnt = nb2 + (s < rem2).astype(jnp.int32)

    def make_a_chunk(lo, hi):
        def a_chunk(t, _):
            ck = a_start + t
            pltpu.sync_copy(ei.at[0, pl.ds(ck * CK, CK)], src_v)
            pltpu.sync_copy(ei.at[1, pl.ds(ck * CK, CK)], dst_v)

            def lane(i, _):
                sv = src_v[pl.ds(i * 16, 16)]
                dv = dst_v[pl.ds(i * 16, 16)]
                s1 = jnp.where(sv != dv, sv, NP)
                s2 = jnp.where(s1 >= lo, s1, NP)
                s3 = jnp.where(s2 < hi, s2, NP)
                aidx_v[pl.ds(i * 16, 16)] = jnp.where(
                    s3 < NP, s3 * NP + dv, AFLAT)
                return _
            lax.fori_loop(0, CK // 16, lane, None)
            pltpu.sync_copy(ones_v, a_out.at[aidx_v])
            return _
        return a_chunk

    @pl.when(c == 0)
    def _():
        lax.fori_loop(0, a_cnt, make_a_chunk(0, HALF), None)

    @pl.when(c == 1)
    def _():
        lax.fori_loop(0, a_cnt, make_a_chunk(HALF, NP), None)

def _sc_edges(edge_index):
    k = pl.kernel(
        _sc_edges_body,
        out_type=(
            jax.ShapeDtypeStruct((APAD,), jnp.float32),
            jax.ShapeDtypeStruct((32 * NPH,), jnp.float32),
            jax.ShapeDtypeStruct((32 * NPH,), jnp.float32),
        ),
        mesh=_mesh(),
        compiler_params=pltpu.CompilerParams(needs_layout_passes=False),
        scratch_types=[
            pltpu.VMEM((CK,), jnp.int32),
            pltpu.VMEM((CK,), jnp.int32),
            pltpu.VMEM((CK,), jnp.int32),
            pltpu.VMEM((CK,), jnp.float32),
            pltpu.VMEM((16384,), jnp.float32),
            pltpu.VMEM((NPH,), jnp.float32),
            pltpu.VMEM((NPH,), jnp.float32),
            pltpu.SemaphoreType.DMA,
        ],
    )
    return k(edge_index)


FW = 128
HA = HALF + 8


def _sc_prop_body(ei, hs, agg,
                  src_v, dst_v, didx_v, gath_v, zer_v, acc_sh, sem):
    c = lax.axis_index("c")
    s = lax.axis_index("s")

    @pl.when(s == 0)
    def _():
        _fill_zero_2d(zer_v, ZR, FW)
        def zbody(i, _):
            pltpu.sync_copy(zer_v, acc_sh.at[pl.ds(i * ZR, ZR)])
            return _
        lax.fori_loop(0, HALF // ZR, zbody, None)
        pltpu.sync_copy(zer_v.at[pl.ds(0, HA - HALF)],
                        acc_sh.at[pl.ds(HALF, HA - HALF)])

    plsc.subcore_barrier()

    nb = NCK // 16
    rem = NCK - nb * 16
    start = s * nb + jnp.minimum(s, rem)
    cnt = nb + (s < rem).astype(jnp.int32)

    def make_chunk(lo, hi):
        def chunk(t, _):
            ck = start + t
            pltpu.sync_copy(ei.at[0, pl.ds(ck * CK, CK)], src_v)
            pltpu.sync_copy(ei.at[1, pl.ds(ck * CK, CK)], dst_v)

            def lane(i, _):
                sv = src_v[pl.ds(i * 16, 16)]
                dv = dst_v[pl.ds(i * 16, 16)]
                t1 = jnp.where(sv != dv, dv, hi)
                t2 = jnp.where(t1 < hi, t1, hi)
                didx_v[pl.ds(i * 16, 16)] = jnp.where(t2 >= lo, t2 - lo, HALF)
                return _
            lax.fori_loop(0, CK // 16, lane, None)
            pltpu.async_copy(hs.at[src_v], gath_v, sem).wait()
            pltpu.sync_copy(gath_v, acc_sh.at[didx_v], add=True)
            return _
        return chunk

    @pl.when(c == 0)
    def _():
        lax.fori_loop(0, cnt, make_chunk(0, HALF), None)

    @pl.when(c == 1)
    def _():
        lax.fori_loop(0, cnt, make_chunk(HALF, NP), None)

    plsc.subcore_barrier()

    @pl.when(s == 0)
    def _():
        pltpu.sync_copy(acc_sh, agg.at[c])


def _sc_prop(edge_index, hs):
    assert hs.shape == (NP, FW)
    k = pl.kernel(
        _sc_prop_body,
        out_type=jax.ShapeDtypeStruct((2, HA, FW), jnp.float32),
        mesh=_mesh(),
        scratch_types=[
            pltpu.VMEM((CK,), jnp.int32),
            pltpu.VMEM((CK,), jnp.int32),
            pltpu.VMEM((CK,), jnp.int32),
            pltpu.VMEM((CK, FW), jnp.float32),
            pltpu.VMEM((ZR, FW), jnp.float32),
            pltpu.VMEM_SHARED((HA, FW), jnp.float32),
            pltpu.SemaphoreType.DMA,
        ],
    )
    return k(edge_index, hs)


def _cvt_body(a_ref, o_ref):
    o_ref[...] = a_ref[...].astype(jnp.bfloat16)


def _to_bf16(a2d, bm=256):
    return pl.pallas_call(
        _cvt_body,
        grid=(NP // bm,),
        in_specs=[pl.BlockSpec((bm, NP), lambda i: (i, 0))],
        out_specs=pl.BlockSpec((bm, NP), lambda i: (i, 0)),
        out_shape=jax.ShapeDtypeStruct((NP, NP), jnp.bfloat16),
    )(a2d)


def _mm_body(al_ref, ar_ref, ad_ref, t_ref, deg_ref, acc, *, bm, bk):
    j = pl.program_id(0)
    i = pl.program_id(1)
    k = pl.program_id(2)

    @pl.when(jnp.logical_and(jnp.logical_and(j == 0, i == 0), k == 0))
    def _():
        deg_ref[...] = jnp.zeros_like(deg_ref)

    @pl.when(k == 0)
    def _():
        acc[...] = jnp.zeros_like(acc)

    acc[...] = acc[...] + jnp.dot(al_ref[...], ar_ref[...],
                                  preferred_element_type=jnp.float32)

    @pl.when(k == pl.num_programs(2) - 1)
    def _():
        rowg = i * bm + lax.broadcasted_iota(jnp.int32, (bm, bm), 0)
        colg = j * bm + lax.broadcasted_iota(jnp.int32, (bm, bm), 1)
        t = jnp.logical_and(jnp.logical_and(acc[...] > 0.0, rowg != colg),
                            ad_ref[...] == 0)
        t_ref[...] = t.astype(jnp.bfloat16)
        deg_ref[0:1, pl.ds(j * bm, bm)] = (
            deg_ref[0:1, pl.ds(j * bm, bm)]
            + jnp.sum(t.astype(jnp.float32), axis=0, keepdims=True))


def _two_hop(abf, bm=1024, bk=1024):
    nm = NP // bm
    nk = NP // bk
    return pl.pallas_call(
        functools.partial(_mm_body, bm=bm, bk=bk),
        grid=(nm, nm, nk),
        in_specs=[
            pl.BlockSpec((bm, bk), lambda j, i, k: (i, k)),
            pl.BlockSpec((bk, bm), lambda j, i, k: (k, j)),
            pl.BlockSpec((bm, bm), lambda j, i, k: (i, j)),
        ],
        out_specs=[
            pl.BlockSpec((bm, bm), lambda j, i, k: (i, j)),
            pl.BlockSpec((8, NP), lambda j, i, k: (0, 0)),
        ],
        out_shape=[
            jax.ShapeDtypeStruct((NP, NP), jnp.bfloat16),
            jax.ShapeDtypeStruct((8, NP), jnp.float32),
        ],
        scratch_shapes=[pltpu.VMEM((bm, bm), jnp.float32)],
    )(abf, abf, abf)


def _dense_prop_body(t_ref, u_ref, o_ref):
    k = pl.program_id(1)

    @pl.when(k == 0)
    def _():
        o_ref[...] = jnp.zeros_like(o_ref)

    o_ref[...] = o_ref[...] + lax.dot_general(
        t_ref[...].astype(jnp.float32), u_ref[...],
        (((0,), (0,)), ((), ())), preferred_element_type=jnp.float32)


def _dense_prop(tmat, u, bm=2048):
    nm = NP // bm
    F = u.shape[1]
    return pl.pallas_call(
        _dense_prop_body,
        grid=(nm, nm),
        in_specs=[
            pl.BlockSpec((bm, bm), lambda j, k: (k, j)),
            pl.BlockSpec((bm, F), lambda j, k: (k, 0)),
        ],
        out_specs=pl.BlockSpec((bm, F), lambda j, k: (j, 0)),
        out_shape=jax.ShapeDtypeStruct((NP, F), jnp.float32),
    )(tmat, u)


def _norms(d1_blk, sc_blk, dt_blk):
    deg1 = jnp.sum(d1_blk, axis=1, keepdims=True)
    dis1 = jnp.where(deg1 > 0.0, lax.rsqrt(deg1), 0.0)
    degt = dt_blk[:, 0:1]
    dis2 = jnp.where(degt > 0.0, lax.rsqrt(degt), 0.0)
    ident = (jnp.sum(sc_blk, axis=1, keepdims=True) == 0.0).astype(jnp.float32)
    return dis1, dis2, ident


def _feat_body(x_ref, w1_ref, d1_ref, sc_ref, dt_ref, h_ref, hs1_ref, hs2_ref):
    h = jnp.dot(x_ref[...], w1_ref[...], preferred_element_type=jnp.float32)
    dis1, dis2, _ = _norms(d1_ref[...], sc_ref[...], dt_ref[...])
    z = jnp.zeros_like(h)
    h_ref[...] = h
    hs1_ref[...] = jnp.concatenate([h * dis1, z], axis=1)
    hs2_ref[...] = h * dis2


def _feat(xp, W1, deg1_t, self_t, degt_t, bm=1024):
    nm = NP // bm
    return pl.pallas_call(
        _feat_body,
        grid=(nm,),
        in_specs=[
            pl.BlockSpec((bm, 128), lambda i: (i, 0)),
            pl.BlockSpec((128, 64), lambda i: (0, 0)),
            pl.BlockSpec((bm, 32), lambda i: (i, 0)),
            pl.BlockSpec((bm, 32), lambda i: (i, 0)),
            pl.BlockSpec((bm, 1), lambda i: (i, 0)),
        ],
        out_specs=[
            pl.BlockSpec((bm, 64), lambda i: (i, 0)),
            pl.BlockSpec((bm, FW), lambda i: (i, 0)),
            pl.BlockSpec((bm, 64), lambda i: (i, 0)),
        ],
        out_shape=[
            jax.ShapeDtypeStruct((NP, 64), jnp.float32),
            jax.ShapeDtypeStruct((NP, FW), jnp.float32),
            jax.ShapeDtypeStruct((NP, 64), jnp.float32),
        ],
    )(xp, W1, deg1_t, self_t, degt_t)


def _layer1_body(a0_ref, a2_ref, h_ref, d1_ref, sc_ref, dt_ref,
                 b1_ref, w2_ref, g_ref, gs1_ref, gs2_ref):
    dis1, dis2, ident = _norms(d1_ref[...], sc_ref[...], dt_ref[...])
    b1 = b1_ref[0:1, :]
    h1 = dis1 * a0_ref[...] + b1
    h12 = dis2 * a2_ref[...] + b1
    h13 = ident * h_ref[...] + b1
    r1 = jax.nn.relu(jnp.concatenate([h1, h12, h13], axis=1))
    g = jnp.dot(r1, w2_ref[...], preferred_element_type=jnp.float32)
    z = jnp.zeros((g.shape[0], FW - 16), jnp.float32)
    g_ref[...] = g
    gs1_ref[...] = jnp.concatenate([g * dis1, z], axis=1)
    gs2_ref[...] = g * dis2


def _layer1(a0, a2, h, deg1_t, self_t, degt_t, b1r, W2, bm=1024):
    nm = NP // bm
    return pl.pallas_call(
        _layer1_body,
        grid=(nm,),
        in_specs=[
            pl.BlockSpec((bm, 64), lambda i: (i, 0)),
            pl.BlockSpec((bm, 64), lambda i: (i, 0)),
            pl.BlockSpec((bm, 64), lambda i: (i, 0)),
            pl.BlockSpec((bm, 32), lambda i: (i, 0)),
            pl.BlockSpec((bm, 32), lambda i: (i, 0)),
            pl.BlockSpec((bm, 1), lambda i: (i, 0)),
            pl.BlockSpec((1, 64), lambda i: (0, 0)),
            pl.BlockSpec((192, 16), lambda i: (0, 0)),
        ],
        out_specs=[
            pl.BlockSpec((bm, 16), lambda i: (i, 0)),
            pl.BlockSpec((bm, FW), lambda i: (i, 0)),
            pl.BlockSpec((bm, 16), lambda i: (i, 0)),
        ],
        out_shape=[
            jax.ShapeDtypeStruct((NP, 16), jnp.float32),
            jax.ShapeDtypeStruct((NP, FW), jnp.float32),
            jax.ShapeDtypeStruct((NP, 16), jnp.float32),
        ],
    )(a0, a2, h, deg1_t, self_t, degt_t, b1r, W2)


def _final_body(p0_ref, p2_ref, g_ref, d1_ref, sc_ref, dt_ref,
                b2_ref, wl_ref, bl_ref, o_ref):
    dis1, dis2, ident = _norms(d1_ref[...], sc_ref[...], dt_ref[...])
    b2 = b2_ref[0:1, :]
    h2 = dis1 * p0_ref[...] + b2
    h22 = dis2 * p2_ref[...] + b2
    h23 = ident * g_ref[...] + b2
    r2 = jnp.concatenate([h2, h22, h23], axis=1)
    fh = jnp.dot(r2, wl_ref[...], preferred_element_type=jnp.float32)
    fh = fh + bl_ref[0:1, :]
    m = jnp.max(fh, axis=1, keepdims=True)
    lse = m + jnp.log(jnp.sum(jnp.exp(fh - m), axis=1, keepdims=True))
    o_ref[...] = fh - lse


def _final(p0, p2, g, deg1_t, self_t, degt_t, b2r, Wlin, blinr, bm=1024):
    nm = NP // bm
    return pl.pallas_call(
        _final_body,
        grid=(nm,),
        in_specs=[
            pl.BlockSpec((bm, 16), lambda i: (i, 0)),
            pl.BlockSpec((bm, 16), lambda i: (i, 0)),
            pl.BlockSpec((bm, 16), lambda i: (i, 0)),
            pl.BlockSpec((bm, 32), lambda i: (i, 0)),
            pl.BlockSpec((bm, 32), lambda i: (i, 0)),
            pl.BlockSpec((bm, 1), lambda i: (i, 0)),
            pl.BlockSpec((1, 16), lambda i: (0, 0)),
            pl.BlockSpec((48, 16), lambda i: (0, 0)),
            pl.BlockSpec((1, 16), lambda i: (0, 0)),
        ],
        out_specs=pl.BlockSpec((bm, 16), lambda i: (i, 0)),
        out_shape=jax.ShapeDtypeStruct((NP, 16), jnp.float32),
    )(p0, p2, g, deg1_t, self_t, degt_t, b2r, Wlin, blinr)


def kernel(x, edge_index, W1, b1, W2, b2, Wlin, blin):
    xp = jnp.concatenate(
        [x, jnp.zeros((NP - N, x.shape[1]), x.dtype)], axis=0)

    aflat, degp, selfp = _sc_edges(edge_index)

    a2d = aflat[:AFLAT].reshape(NP, NP)
    abf = _to_bf16(a2d)
    tmat, degt = _two_hop(abf)

    deg1_t = degp.reshape(32, NPH)[:, :NP].T
    self_t = selfp.reshape(32, NPH)[:, :NP].T
    degt_t = degt[0:1, :].T

    h, hs1, hs2 = _feat(xp, W1, deg1_t, self_t, degt_t)

    agg1 = _sc_prop(edge_index, hs1)
    a1full = jnp.concatenate([agg1[0, :HALF, :64], agg1[1, :HALF, :64]], 0)
    agg2 = _dense_prop(tmat, hs2)

    g, gs1, gs2 = _layer1(a1full, agg2, h,
                          deg1_t, self_t, degt_t,
                          b1.reshape(1, 64), W2)

    bgg1 = _sc_prop(edge_index, gs1)
    b1full = jnp.concatenate([bgg1[0, :HALF, :16], bgg1[1, :HALF, :16]], 0)
    bgg2 = _dense_prop(tmat, gs2)

    out = _final(b1full, bgg2, g,
                 deg1_t, self_t, degt_t,
                 b2.reshape(1, 16), Wlin, blin.reshape(1, 16))
    return out[:N]


# speedup vs baseline: 5.0293x; 5.0293x over previous
import functools

import jax
import jax.numpy as jnp
from jax import lax
from jax.experimental import pallas as pl
from jax.experimental.pallas import tpu as pltpu
from jax.experimental.pallas import tpu_sc as plsc

N = 10000
NP = 10240
NPD = NP + 8
E = 160000
CK = 128
NCK = E // CK
HALF = NP // 2
AFLAT = NP * NP
APAD = AFLAT + 8
ZR = 256
NPH = NP + 16


def _mesh():
    return plsc.VectorSubcoreMesh(core_axis_name="c", subcore_axis_name="s")


def _fill_zero_1d(ref, n):
    def body(i, _):
        ref[pl.ds(i * 16, 16)] = jnp.zeros((16,), ref.dtype)
        return _
    lax.fori_loop(0, n // 16, body, None)


def _fill_zero_2d(ref, rows, cols):
    def body(i, _):
        r = i // (cols // 16)
        c = (i % (cols // 16)) * 16
        ref[r, pl.ds(c, 16)] = jnp.zeros((16,), ref.dtype)
        return _
    lax.fori_loop(0, rows * (cols // 16), body, None)


EPT = E // 16
SW = 128
NSTR = NP // SW
QROW = HALF // 2
SROW = QROW // 16
SH_ROWS = QROW + 8


def _sc_edges_body(ei, a3_out, degp, selfp,
                   esrc_v, edst_v, lsv_v, ldv_v, oh_v, ridx_v, zer_v,
                   deg_v, self_v, stripe_sh, sem):
    c = lax.axis_index("c")
    s = lax.axis_index("s")
    wid = c * 16 + s

    _fill_zero_2d(zer_v, SW, SW)
    _fill_zero_1d(deg_v, NPH)
    _fill_zero_1d(self_v, NPH)
    _fill_zero_2d(oh_v, CK, SW)
    ones16 = jnp.ones((16,), jnp.float32)
    iota16 = lax.iota(jnp.int32, 16)

    pltpu.sync_copy(ei.at[pl.ds(s * EPT, EPT)], esrc_v)
    pltpu.sync_copy(ei.at[pl.ds(E + s * EPT, EPT)], edst_v)

    h_lo = c * 312
    h_n = 312 + c

    def hist_lane(i, _):
        o = (h_lo + i) * 16
        sv = esrc_v[pl.ds(o, 16)]
        dv = edst_v[pl.ds(o, 16)]
        didx = jnp.where(sv != dv, dv, NP)
        sidx = jnp.where(sv == dv, sv, NP)
        plsc.addupdate_scatter(deg_v, [didx], ones16)
        plsc.addupdate_scatter(self_v, [sidx], ones16)
        return _
    lax.fori_loop(0, h_n, hist_lane, None)

    pltpu.sync_copy(deg_v, degp.at[pl.ds(wid * NPH, NPH)])
    pltpu.sync_copy(self_v, selfp.at[pl.ds(wid * NPH, NPH)])

    rbase = c * HALF

    def stripe(t, _):
        p = t // 2
        clo = p * SW
        qbase = rbase + (t % 2) * QROW
        pltpu.sync_copy(zer_v, stripe_sh.at[pl.ds(s * SROW, SW)])
        pltpu.sync_copy(zer_v.at[pl.ds(0, SROW - SW)],
                        stripe_sh.at[pl.ds(s * SROW + SW, SROW - SW)])
        @pl.when(s == 0)
        def _():
            pltpu.sync_copy(zer_v.at[pl.ds(0, SH_ROWS - QROW)],
                            stripe_sh.at[pl.ds(QROW, SH_ROWS - QROW)])
        plsc.subcore_barrier()

        def scan(i, off):
            sv = esrc_v[pl.ds(i * 16, 16)]
            dv = edst_v[pl.ds(i * 16, 16)]
            m1 = jnp.where(sv != dv, 1, 0)
            m2 = jnp.where(sv >= qbase, 1, 0)
            m3 = jnp.where(sv < qbase + QROW, 1, 0)
            m4 = jnp.where(dv >= clo, 1, 0)
            m5 = jnp.where(dv < clo + SW, 1, 0)
            mask = (m1 + m2 + m3 + m4 + m5) == 5
            plsc.store_compressed(lsv_v.at[pl.ds(off, 16)], sv, mask=mask)
            plsc.store_compressed(ldv_v.at[pl.ds(off, 16)], dv, mask=mask)
            return off + jnp.sum(jnp.where(mask, 1, 0))
        cnt = lax.fori_loop(0, EPT // 16, scan, jnp.int32(0))

        def batch(b, _):
            e0 = b * CK
            for g in range(CK // 16):
                svg = lsv_v[pl.ds(e0 + g * 16, 16)]
                dvg = ldv_v[pl.ds(e0 + g * 16, 16)]
                lanemask = (e0 + g * 16 + iota16) < cnt
                rowpos = iota16 + g * 16
                dcol = dvg - clo
                plsc.store_scatter(oh_v, [rowpos, dcol], ones16, mask=lanemask)
                ridx_v[pl.ds(g * 16, 16)] = jnp.where(lanemask, svg - qbase,
                                                      QROW)
            pltpu.sync_copy(oh_v, stripe_sh.at[ridx_v], add=True)
            for g in range(CK // 16):
                dvg = ldv_v[pl.ds(e0 + g * 16, 16)]
                lanemask = (e0 + g * 16 + iota16) < cnt
                rowpos = iota16 + g * 16
                dcol = dvg - clo
                plsc.store_scatter(oh_v, [rowpos, dcol],
                                   jnp.zeros((16,), jnp.float32),
                                   mask=lanemask)
            return _
        lax.fori_loop(0, (cnt + CK - 1) // CK, batch, None)

        plsc.subcore_barrier()
        pltpu.sync_copy(stripe_sh.at[pl.ds(s * SROW, SROW)],
                        a3_out.at[pl.ds(p * NP + qbase + s * SROW, SROW)])
        return _
    lax.fori_loop(0, 2 * NSTR, stripe, None)


def _sc_edges(edge_index):
    k = pl.kernel(
        _sc_edges_body,
        out_type=(
            jax.ShapeDtypeStruct((NSTR * NP, SW), jnp.float32),
            jax.ShapeDtypeStruct((32 * NPH,), jnp.float32),
            jax.ShapeDtypeStruct((32 * NPH,), jnp.float32),
        ),
        mesh=_mesh(),
        compiler_params=pltpu.CompilerParams(needs_layout_passes=False),
        scratch_types=[
            pltpu.VMEM((EPT,), jnp.int32),
            pltpu.VMEM((EPT,), jnp.int32),
            pltpu.VMEM((EPT + 16,), jnp.int32),
            pltpu.VMEM((EPT + 16,), jnp.int32),
            pltpu.VMEM((CK, SW), jnp.float32),
            pltpu.VMEM((CK,), jnp.int32),
            pltpu.VMEM((SW, SW), jnp.float32),
            pltpu.VMEM((NPH,), jnp.float32),
            pltpu.VMEM((NPH,), jnp.float32),
            pltpu.VMEM_SHARED((SH_ROWS, SW), jnp.float32),
            pltpu.SemaphoreType.DMA,
        ],
    )
    return k(edge_index.reshape(2 * E))


FW = 128
HA = HALF + 8


def _sc_prop_body(ei, hs, agg,
                  src_v, dst_v, didx_v, gath_v, zer_v, acc_sh, sem):
    c = lax.axis_index("c")
    s = lax.axis_index("s")

    @pl.when(s == 0)
    def _():
        _fill_zero_2d(zer_v, ZR, FW)
        def zbody(i, _):
            pltpu.sync_copy(zer_v, acc_sh.at[pl.ds(i * ZR, ZR)])
            return _
        lax.fori_loop(0, HALF // ZR, zbody, None)
        pltpu.sync_copy(zer_v.at[pl.ds(0, HA - HALF)],
                        acc_sh.at[pl.ds(HALF, HA - HALF)])

    plsc.subcore_barrier()

    nb = NCK // 16
    rem = NCK - nb * 16
    start = s * nb + jnp.minimum(s, rem)
    cnt = nb + (s < rem).astype(jnp.int32)

    def make_chunk(lo, hi):
        def chunk(t, _):
            ck = start + t
            pltpu.sync_copy(ei.at[0, pl.ds(ck * CK, CK)], src_v)
            pltpu.sync_copy(ei.at[1, pl.ds(ck * CK, CK)], dst_v)

            def lane(i, _):
                sv = src_v[pl.ds(i * 16, 16)]
                dv = dst_v[pl.ds(i * 16, 16)]
                t1 = jnp.where(sv != dv, dv, hi)
                t2 = jnp.where(t1 < hi, t1, hi)
                didx_v[pl.ds(i * 16, 16)] = jnp.where(t2 >= lo, t2 - lo, HALF)
                return _
            lax.fori_loop(0, CK // 16, lane, None)
            pltpu.async_copy(hs.at[src_v], gath_v, sem).wait()
            pltpu.sync_copy(gath_v, acc_sh.at[didx_v], add=True)
            return _
        return chunk

    @pl.when(c == 0)
    def _():
        lax.fori_loop(0, cnt, make_chunk(0, HALF), None)

    @pl.when(c == 1)
    def _():
        lax.fori_loop(0, cnt, make_chunk(HALF, NP), None)

    plsc.subcore_barrier()

    @pl.when(s == 0)
    def _():
        pltpu.sync_copy(acc_sh, agg.at[c])


def _sc_prop(edge_index, hs):
    assert hs.shape == (NP, FW)
    k = pl.kernel(
        _sc_prop_body,
        out_type=jax.ShapeDtypeStruct((2, HA, FW), jnp.float32),
        mesh=_mesh(),
        scratch_types=[
            pltpu.VMEM((CK,), jnp.int32),
            pltpu.VMEM((CK,), jnp.int32),
            pltpu.VMEM((CK,), jnp.int32),
            pltpu.VMEM((CK, FW), jnp.float32),
            pltpu.VMEM((ZR, FW), jnp.float32),
            pltpu.VMEM_SHARED((HA, FW), jnp.float32),
            pltpu.SemaphoreType.DMA,
        ],
    )
    return k(edge_index, hs)


def _cvt_body(a_ref, o_ref):
    o_ref[...] = a_ref[...].reshape(o_ref.shape).astype(jnp.bfloat16)


def _to_bf16(a3, bm=1024):
    nm = NP // bm
    return pl.pallas_call(
        _cvt_body,
        grid=(nm, NSTR),
        in_specs=[pl.BlockSpec((bm, SW), lambda i, p: (p * nm + i, 0))],
        out_specs=pl.BlockSpec((bm, SW), lambda i, p: (i, p)),
        out_shape=jax.ShapeDtypeStruct((NP, NP), jnp.bfloat16),
    )(a3)


def _mm_body(al_ref, ar_ref, ad_ref, t_ref, deg_ref, acc, *, bm, bk):
    j = pl.program_id(0)
    i = pl.program_id(1)
    k = pl.program_id(2)

    @pl.when(jnp.logical_and(jnp.logical_and(j == 0, i == 0), k == 0))
    def _():
        deg_ref[...] = jnp.zeros_like(deg_ref)

    @pl.when(k == 0)
    def _():
        acc[...] = jnp.zeros_like(acc)

    acc[...] = acc[...] + jnp.dot(al_ref[...], ar_ref[...],
                                  preferred_element_type=jnp.float32)

    @pl.when(k == pl.num_programs(2) - 1)
    def _():
        rowg = i * bm + lax.broadcasted_iota(jnp.int32, (bm, bm), 0)
        colg = j * bm + lax.broadcasted_iota(jnp.int32, (bm, bm), 1)
        t = jnp.logical_and(jnp.logical_and(acc[...] > 0.0, rowg != colg),
                            ad_ref[...] == 0)
        t_ref[...] = t.astype(jnp.bfloat16)
        deg_ref[0:1, pl.ds(j * bm, bm)] = (
            deg_ref[0:1, pl.ds(j * bm, bm)]
            + jnp.sum(t.astype(jnp.float32), axis=0, keepdims=True))


def _two_hop(abf, bm=1024, bk=1024):
    nm = NP // bm
    nk = NP // bk
    return pl.pallas_call(
        functools.partial(_mm_body, bm=bm, bk=bk),
        grid=(nm, nm, nk),
        in_specs=[
            pl.BlockSpec((bm, bk), lambda j, i, k: (i, k)),
            pl.BlockSpec((bk, bm), lambda j, i, k: (k, j)),
            pl.BlockSpec((bm, bm), lambda j, i, k: (i, j)),
        ],
        out_specs=[
            pl.BlockSpec((bm, bm), lambda j, i, k: (i, j)),
            pl.BlockSpec((8, NP), lambda j, i, k: (0, 0)),
        ],
        out_shape=[
            jax.ShapeDtypeStruct((NP, NP), jnp.bfloat16),
            jax.ShapeDtypeStruct((8, NP), jnp.float32),
        ],
        scratch_shapes=[pltpu.VMEM((bm, bm), jnp.float32)],
    )(abf, abf, abf)


def _dense_prop_body(t_ref, u_ref, o_ref):
    k = pl.program_id(1)

    @pl.when(k == 0)
    def _():
        o_ref[...] = jnp.zeros_like(o_ref)

    o_ref[...] = o_ref[...] + lax.dot_general(
        t_ref[...].astype(jnp.float32), u_ref[...],
        (((0,), (0,)), ((), ())), preferred_element_type=jnp.float32)


def _dense_prop(tmat, u, bm=2048):
    nm = NP // bm
    F = u.shape[1]
    return pl.pallas_call(
        _dense_prop_body,
        grid=(nm, nm),
        in_specs=[
            pl.BlockSpec((bm, bm), lambda j, k: (k, j)),
            pl.BlockSpec((bm, F), lambda j, k: (k, 0)),
        ],
        out_specs=pl.BlockSpec((bm, F), lambda j, k: (j, 0)),
        out_shape=jax.ShapeDtypeStruct((NP, F), jnp.float32),
    )(tmat, u)


def _norms(d1_blk, sc_blk, dt_blk):
    deg1 = jnp.sum(d1_blk, axis=1, keepdims=True)
    dis1 = jnp.where(deg1 > 0.0, lax.rsqrt(deg1), 0.0)
    degt = dt_blk[:, 0:1]
    dis2 = jnp.where(degt > 0.0, lax.rsqrt(degt), 0.0)
    ident = (jnp.sum(sc_blk, axis=1, keepdims=True) == 0.0).astype(jnp.float32)
    return dis1, dis2, ident


def _feat_body(x_ref, w1_ref, d1_ref, sc_ref, dt_ref, h_ref, hs1_ref, hs2_ref):
    h = jnp.dot(x_ref[...], w1_ref[...], preferred_element_type=jnp.float32)
    dis1, dis2, _ = _norms(d1_ref[...], sc_ref[...], dt_ref[...])
    z = jnp.zeros_like(h)
    h_ref[...] = h
    hs1_ref[...] = jnp.concatenate([h * dis1, z], axis=1)
    hs2_ref[...] = h * dis2


def _feat(xp, W1, deg1_t, self_t, degt_t, bm=1024):
    nm = NP // bm
    return pl.pallas_call(
        _feat_body,
        grid=(nm,),
        in_specs=[
            pl.BlockSpec((bm, 128), lambda i: (i, 0)),
            pl.BlockSpec((128, 64), lambda i: (0, 0)),
            pl.BlockSpec((bm, 32), lambda i: (i, 0)),
            pl.BlockSpec((bm, 32), lambda i: (i, 0)),
            pl.BlockSpec((bm, 1), lambda i: (i, 0)),
        ],
        out_specs=[
            pl.BlockSpec((bm, 64), lambda i: (i, 0)),
            pl.BlockSpec((bm, FW), lambda i: (i, 0)),
            pl.BlockSpec((bm, 64), lambda i: (i, 0)),
        ],
        out_shape=[
            jax.ShapeDtypeStruct((NP, 64), jnp.float32),
            jax.ShapeDtypeStruct((NP, FW), jnp.float32),
            jax.ShapeDtypeStruct((NP, 64), jnp.float32),
        ],
    )(xp, W1, deg1_t, self_t, degt_t)


def _layer1_body(a0_ref, a2_ref, h_ref, d1_ref, sc_ref, dt_ref,
                 b1_ref, w2_ref, g_ref, gs1_ref, gs2_ref):
    dis1, dis2, ident = _norms(d1_ref[...], sc_ref[...], dt_ref[...])
    b1 = b1_ref[0:1, :]
    h1 = dis1 * a0_ref[...] + b1
    h12 = dis2 * a2_ref[...] + b1
    h13 = ident * h_ref[...] + b1
    r1 = jax.nn.relu(jnp.concatenate([h1, h12, h13], axis=1))
    g = jnp.dot(r1, w2_ref[...], preferred_element_type=jnp.float32)
    z = jnp.zeros((g.shape[0], FW - 16), jnp.float32)
    g_ref[...] = g
    gs1_ref[...] = jnp.concatenate([g * dis1, z], axis=1)
    gs2_ref[...] = g * dis2


def _layer1(a0, a2, h, deg1_t, self_t, degt_t, b1r, W2, bm=1024):
    nm = NP // bm
    return pl.pallas_call(
        _layer1_body,
        grid=(nm,),
        in_specs=[
            pl.BlockSpec((bm, 64), lambda i: (i, 0)),
            pl.BlockSpec((bm, 64), lambda i: (i, 0)),
            pl.BlockSpec((bm, 64), lambda i: (i, 0)),
            pl.BlockSpec((bm, 32), lambda i: (i, 0)),
            pl.BlockSpec((bm, 32), lambda i: (i, 0)),
            pl.BlockSpec((bm, 1), lambda i: (i, 0)),
            pl.BlockSpec((1, 64), lambda i: (0, 0)),
            pl.BlockSpec((192, 16), lambda i: (0, 0)),
        ],
        out_specs=[
            pl.BlockSpec((bm, 16), lambda i: (i, 0)),
            pl.BlockSpec((bm, FW), lambda i: (i, 0)),
            pl.BlockSpec((bm, 16), lambda i: (i, 0)),
        ],
        out_shape=[
            jax.ShapeDtypeStruct((NP, 16), jnp.float32),
            jax.ShapeDtypeStruct((NP, FW), jnp.float32),
            jax.ShapeDtypeStruct((NP, 16), jnp.float32),
        ],
    )(a0, a2, h, deg1_t, self_t, degt_t, b1r, W2)


def _final_body(p0_ref, p2_ref, g_ref, d1_ref, sc_ref, dt_ref,
                b2_ref, wl_ref, bl_ref, o_ref):
    dis1, dis2, ident = _norms(d1_ref[...], sc_ref[...], dt_ref[...])
    b2 = b2_ref[0:1, :]
    h2 = dis1 * p0_ref[...] + b2
    h22 = dis2 * p2_ref[...] + b2
    h23 = ident * g_ref[...] + b2
    r2 = jnp.concatenate([h2, h22, h23], axis=1)
    fh = jnp.dot(r2, wl_ref[...], preferred_element_type=jnp.float32)
    fh = fh + bl_ref[0:1, :]
    m = jnp.max(fh, axis=1, keepdims=True)
    lse = m + jnp.log(jnp.sum(jnp.exp(fh - m), axis=1, keepdims=True))
    o_ref[...] = fh - lse


def _final(p0, p2, g, deg1_t, self_t, degt_t, b2r, Wlin, blinr, bm=1024):
    nm = NP // bm
    return pl.pallas_call(
        _final_body,
        grid=(nm,),
        in_specs=[
            pl.BlockSpec((bm, 16), lambda i: (i, 0)),
            pl.BlockSpec((bm, 16), lambda i: (i, 0)),
            pl.BlockSpec((bm, 16), lambda i: (i, 0)),
            pl.BlockSpec((bm, 32), lambda i: (i, 0)),
            pl.BlockSpec((bm, 32), lambda i: (i, 0)),
            pl.BlockSpec((bm, 1), lambda i: (i, 0)),
            pl.BlockSpec((1, 16), lambda i: (0, 0)),
            pl.BlockSpec((48, 16), lambda i: (0, 0)),
            pl.BlockSpec((1, 16), lambda i: (0, 0)),
        ],
        out_specs=pl.BlockSpec((bm, 16), lambda i: (i, 0)),
        out_shape=jax.ShapeDtypeStruct((NP, 16), jnp.float32),
    )(p0, p2, g, deg1_t, self_t, degt_t, b2r, Wlin, blinr)


def kernel(x, edge_index, W1, b1, W2, b2, Wlin, blin):
    xp = jnp.concatenate(
        [x, jnp.zeros((NP - N, x.shape[1]), x.dtype)], axis=0)

    a3, degp, selfp = _sc_edges(edge_index)
    abf = _to_bf16(a3)
    tmat, degt = _two_hop(abf)

    deg1_t = degp.reshape(32, NPH)[:, :NP].T
    self_t = selfp.reshape(32, NPH)[:, :NP].T
    degt_t = degt[0:1, :].T

    h, hs1, hs2 = _feat(xp, W1, deg1_t, self_t, degt_t)

    agg1 = _sc_prop(edge_index, hs1)
    a1full = jnp.concatenate([agg1[0, :HALF, :64], agg1[1, :HALF, :64]], 0)
    agg2 = _dense_prop(tmat, hs2)

    g, gs1, gs2 = _layer1(a1full, agg2, h,
                          deg1_t, self_t, degt_t,
                          b1.reshape(1, 64), W2)

    bgg1 = _sc_prop(edge_index, gs1)
    b1full = jnp.concatenate([bgg1[0, :HALF, :16], bgg1[1, :HALF, :16]], 0)
    bgg2 = _dense_prop(tmat, gs2)

    out = _final(b1full, bgg2, g,
                 deg1_t, self_t, degt_t,
                 b2.reshape(1, 16), Wlin, blin.reshape(1, 16))
    return out[:N]


# speedup vs baseline: 5.1636x; 1.0267x over previous
import functools

import jax
import jax.numpy as jnp
from jax import lax
from jax.experimental import pallas as pl
from jax.experimental.pallas import tpu as pltpu
from jax.experimental.pallas import tpu_sc as plsc

N = 10000
NP = 10240
NPD = NP + 8
E = 160000
CK = 128
NCK = E // CK
HALF = NP // 2
AFLAT = NP * NP
APAD = AFLAT + 8
ZR = 256
NPH = NP + 16


def _mesh():
    return plsc.VectorSubcoreMesh(core_axis_name="c", subcore_axis_name="s")


def _fill_zero_1d(ref, n):
    def body(i, _):
        ref[pl.ds(i * 16, 16)] = jnp.zeros((16,), ref.dtype)
        return _
    lax.fori_loop(0, n // 16, body, None)


def _fill_zero_2d(ref, rows, cols):
    def body(i, _):
        r = i // (cols // 16)
        c = (i % (cols // 16)) * 16
        ref[r, pl.ds(c, 16)] = jnp.zeros((16,), ref.dtype)
        return _
    lax.fori_loop(0, rows * (cols // 16), body, None)


EPT = E // 16
SW = 128
NSTR = NP // SW
QROW = HALF // 2
SROW = QROW // 16
SH_ROWS = QROW + 8


def _sc_edges_body(ei, a3_out, degp, selfp,
                   esrc_v, edst_v, lsv_v, ldv_v, oh_v, ridx_v, zer_v,
                   deg_v, self_v, stripe_sh, sem):
    c = lax.axis_index("c")
    s = lax.axis_index("s")
    wid = c * 16 + s

    _fill_zero_2d(zer_v, SW, SW)
    _fill_zero_1d(deg_v, NPH)
    _fill_zero_1d(self_v, NPH)
    _fill_zero_2d(oh_v, CK, SW)
    ones16 = jnp.ones((16,), jnp.float32)
    iota16 = lax.iota(jnp.int32, 16)

    pltpu.sync_copy(ei.at[pl.ds(s * EPT, EPT)], esrc_v)
    pltpu.sync_copy(ei.at[pl.ds(E + s * EPT, EPT)], edst_v)

    h_lo = c * 312
    h_n = 312 + c

    def hist_lane(i, _):
        o = (h_lo + i) * 16
        sv = esrc_v[pl.ds(o, 16)]
        dv = edst_v[pl.ds(o, 16)]
        didx = jnp.where(sv != dv, dv, NP)
        sidx = jnp.where(sv == dv, sv, NP)
        plsc.addupdate_scatter(deg_v, [didx], ones16)
        plsc.addupdate_scatter(self_v, [sidx], ones16)
        return _
    lax.fori_loop(0, h_n, hist_lane, None)

    pltpu.sync_copy(deg_v, degp.at[pl.ds(wid * NPH, NPH)])
    pltpu.sync_copy(self_v, selfp.at[pl.ds(wid * NPH, NPH)])

    rbase = c * HALF

    def stripe(t, _):
        p = t // 2
        clo = p * SW
        qbase = rbase + (t % 2) * QROW
        pltpu.sync_copy(zer_v, stripe_sh.at[pl.ds(s * SROW, SW)])
        pltpu.sync_copy(zer_v.at[pl.ds(0, SROW - SW)],
                        stripe_sh.at[pl.ds(s * SROW + SW, SROW - SW)])
        @pl.when(s == 0)
        def _():
            pltpu.sync_copy(zer_v.at[pl.ds(0, SH_ROWS - QROW)],
                            stripe_sh.at[pl.ds(QROW, SH_ROWS - QROW)])
        plsc.subcore_barrier()

        def scan(i, off):
            sv = esrc_v[pl.ds(i * 16, 16)]
            dv = edst_v[pl.ds(i * 16, 16)]
            m1 = jnp.where(sv != dv, 1, 0)
            m2 = jnp.where(sv >= qbase, 1, 0)
            m3 = jnp.where(sv < qbase + QROW, 1, 0)
            m4 = jnp.where(dv >= clo, 1, 0)
            m5 = jnp.where(dv < clo + SW, 1, 0)
            mask = (m1 + m2 + m3 + m4 + m5) == 5
            plsc.store_compressed(lsv_v.at[pl.ds(off, 16)], sv, mask=mask)
            plsc.store_compressed(ldv_v.at[pl.ds(off, 16)], dv, mask=mask)
            return off + jnp.sum(jnp.where(mask, 1, 0))
        cnt = lax.fori_loop(0, EPT // 16, scan, jnp.int32(0))

        def batch(b, _):
            e0 = b * CK
            for g in range(CK // 16):
                svg = lsv_v[pl.ds(e0 + g * 16, 16)]
                dvg = ldv_v[pl.ds(e0 + g * 16, 16)]
                lanemask = (e0 + g * 16 + iota16) < cnt
                rowpos = iota16 + g * 16
                dcol = dvg - clo
                plsc.store_scatter(oh_v, [rowpos, dcol], ones16, mask=lanemask)
                ridx_v[pl.ds(g * 16, 16)] = jnp.where(lanemask, svg - qbase,
                                                      QROW)
            pltpu.sync_copy(oh_v, stripe_sh.at[ridx_v], add=True)
            for g in range(CK // 16):
                dvg = ldv_v[pl.ds(e0 + g * 16, 16)]
                lanemask = (e0 + g * 16 + iota16) < cnt
                rowpos = iota16 + g * 16
                dcol = dvg - clo
                plsc.store_scatter(oh_v, [rowpos, dcol],
                                   jnp.zeros((16,), jnp.float32),
                                   mask=lanemask)
            return _
        lax.fori_loop(0, (cnt + CK - 1) // CK, batch, None)

        plsc.subcore_barrier()
        pltpu.sync_copy(stripe_sh.at[pl.ds(s * SROW, SROW)],
                        a3_out.at[pl.ds(p * NP + qbase + s * SROW, SROW)])
        return _
    lax.fori_loop(0, 2 * NSTR, stripe, None)


def _sc_edges(edge_index):
    k = pl.kernel(
        _sc_edges_body,
        out_type=(
            jax.ShapeDtypeStruct((NSTR * NP, SW), jnp.float32),
            jax.ShapeDtypeStruct((32 * NPH,), jnp.float32),
            jax.ShapeDtypeStruct((32 * NPH,), jnp.float32),
        ),
        mesh=_mesh(),
        compiler_params=pltpu.CompilerParams(needs_layout_passes=False),
        scratch_types=[
            pltpu.VMEM((EPT,), jnp.int32),
            pltpu.VMEM((EPT,), jnp.int32),
            pltpu.VMEM((EPT + 16,), jnp.int32),
            pltpu.VMEM((EPT + 16,), jnp.int32),
            pltpu.VMEM((CK, SW), jnp.float32),
            pltpu.VMEM((CK,), jnp.int32),
            pltpu.VMEM((SW, SW), jnp.float32),
            pltpu.VMEM((NPH,), jnp.float32),
            pltpu.VMEM((NPH,), jnp.float32),
            pltpu.VMEM_SHARED((SH_ROWS, SW), jnp.float32),
            pltpu.SemaphoreType.DMA,
        ],
    )
    return k(edge_index.reshape(2 * E))


FW = 128
HA = HALF + 8


def _sc_prop_body(ei, hs, agg,
                  src_v, dst_v, didx_v, gath_v, zer_v, acc_sh, sem):
    c = lax.axis_index("c")
    s = lax.axis_index("s")

    @pl.when(s == 0)
    def _():
        _fill_zero_2d(zer_v, ZR, FW)
        def zbody(i, _):
            pltpu.sync_copy(zer_v, acc_sh.at[pl.ds(i * ZR, ZR)])
            return _
        lax.fori_loop(0, HALF // ZR, zbody, None)
        pltpu.sync_copy(zer_v.at[pl.ds(0, HA - HALF)],
                        acc_sh.at[pl.ds(HALF, HA - HALF)])

    plsc.subcore_barrier()

    nb = NCK // 16
    rem = NCK - nb * 16
    start = s * nb + jnp.minimum(s, rem)
    cnt = nb + (s < rem).astype(jnp.int32)

    def make_chunk(lo, hi):
        def chunk(t, _):
            ck = start + t
            pltpu.sync_copy(ei.at[0, pl.ds(ck * CK, CK)], src_v)
            pltpu.sync_copy(ei.at[1, pl.ds(ck * CK, CK)], dst_v)

            def lane(i, _):
                sv = src_v[pl.ds(i * 16, 16)]
                dv = dst_v[pl.ds(i * 16, 16)]
                t1 = jnp.where(sv != dv, dv, hi)
                t2 = jnp.where(t1 < hi, t1, hi)
                didx_v[pl.ds(i * 16, 16)] = jnp.where(t2 >= lo, t2 - lo, HALF)
                return _
            lax.fori_loop(0, CK // 16, lane, None)
            pltpu.async_copy(hs.at[src_v], gath_v, sem).wait()
            pltpu.sync_copy(gath_v, acc_sh.at[didx_v], add=True)
            return _
        return chunk

    @pl.when(c == 0)
    def _():
        lax.fori_loop(0, cnt, make_chunk(0, HALF), None)

    @pl.when(c == 1)
    def _():
        lax.fori_loop(0, cnt, make_chunk(HALF, NP), None)

    plsc.subcore_barrier()

    @pl.when(s == 0)
    def _():
        pltpu.sync_copy(acc_sh, agg.at[c])


def _sc_prop(edge_index, hs):
    assert hs.shape == (NP, FW)
    k = pl.kernel(
        _sc_prop_body,
        out_type=jax.ShapeDtypeStruct((2, HA, FW), jnp.float32),
        mesh=_mesh(),
        scratch_types=[
            pltpu.VMEM((CK,), jnp.int32),
            pltpu.VMEM((CK,), jnp.int32),
            pltpu.VMEM((CK,), jnp.int32),
            pltpu.VMEM((CK, FW), jnp.float32),
            pltpu.VMEM((ZR, FW), jnp.float32),
            pltpu.VMEM_SHARED((HA, FW), jnp.float32),
            pltpu.SemaphoreType.DMA,
        ],
    )
    return k(edge_index, hs)


def _cvt_body(a_ref, o_ref):
    o_ref[...] = a_ref[...].reshape(o_ref.shape).astype(jnp.bfloat16)


def _to_bf16(a3, bm=1024):
    nm = NP // bm
    return pl.pallas_call(
        _cvt_body,
        grid=(nm, NSTR),
        in_specs=[pl.BlockSpec((bm, SW), lambda i, p: (p * nm + i, 0))],
        out_specs=pl.BlockSpec((bm, SW), lambda i, p: (i, p)),
        out_shape=jax.ShapeDtypeStruct((NP, NP), jnp.bfloat16),
    )(a3)


def _mm_body(al_ref, ar_ref, ad_ref, t_ref, deg_ref, acc, *, bm, bk):
    j = pl.program_id(0)
    i = pl.program_id(1)
    k = pl.program_id(2)

    @pl.when(jnp.logical_and(jnp.logical_and(j == 0, i == 0), k == 0))
    def _():
        deg_ref[...] = jnp.zeros_like(deg_ref)

    @pl.when(k == 0)
    def _():
        acc[...] = jnp.zeros_like(acc)

    acc[...] = acc[...] + jnp.dot(al_ref[...], ar_ref[...],
                                  preferred_element_type=jnp.float32)

    @pl.when(k == pl.num_programs(2) - 1)
    def _():
        rowg = i * bm + lax.broadcasted_iota(jnp.int32, (bm, bm), 0)
        colg = j * bm + lax.broadcasted_iota(jnp.int32, (bm, bm), 1)
        t = jnp.logical_and(jnp.logical_and(acc[...] > 0.0, rowg != colg),
                            ad_ref[...] == 0)
        t_ref[...] = t.astype(jnp.bfloat16)
        deg_ref[0:1, pl.ds(j * bm, bm)] = (
            deg_ref[0:1, pl.ds(j * bm, bm)]
            + jnp.sum(t.astype(jnp.float32), axis=0, keepdims=True))


def _two_hop(abf, bm=2048, bk=512):
    nm = NP // bm
    nk = NP // bk
    return pl.pallas_call(
        functools.partial(_mm_body, bm=bm, bk=bk),
        grid=(nm, nm, nk),
        in_specs=[
            pl.BlockSpec((bm, bk), lambda j, i, k: (i, k)),
            pl.BlockSpec((bk, bm), lambda j, i, k: (k, j)),
            pl.BlockSpec((bm, bm), lambda j, i, k: (i, j)),
        ],
        out_specs=[
            pl.BlockSpec((bm, bm), lambda j, i, k: (i, j)),
            pl.BlockSpec((8, NP), lambda j, i, k: (0, 0)),
        ],
        out_shape=[
            jax.ShapeDtypeStruct((NP, NP), jnp.bfloat16),
            jax.ShapeDtypeStruct((8, NP), jnp.float32),
        ],
        scratch_shapes=[pltpu.VMEM((bm, bm), jnp.float32)],
    )(abf, abf, abf)


def _dense_prop_body(t_ref, u_ref, o_ref):
    k = pl.program_id(1)

    @pl.when(k == 0)
    def _():
        o_ref[...] = jnp.zeros_like(o_ref)

    o_ref[...] = o_ref[...] + lax.dot_general(
        t_ref[...].astype(jnp.float32), u_ref[...],
        (((0,), (0,)), ((), ())), preferred_element_type=jnp.float32)


def _dense_prop(tmat, u, bm=2048):
    nm = NP // bm
    F = u.shape[1]
    return pl.pallas_call(
        _dense_prop_body,
        grid=(nm, nm),
        in_specs=[
            pl.BlockSpec((bm, bm), lambda j, k: (k, j)),
            pl.BlockSpec((bm, F), lambda j, k: (k, 0)),
        ],
        out_specs=pl.BlockSpec((bm, F), lambda j, k: (j, 0)),
        out_shape=jax.ShapeDtypeStruct((NP, F), jnp.float32),
    )(tmat, u)


def _norms(d1_blk, sc_blk, dt_blk):
    deg1 = jnp.sum(d1_blk, axis=1, keepdims=True)
    dis1 = jnp.where(deg1 > 0.0, lax.rsqrt(deg1), 0.0)
    degt = dt_blk[:, 0:1]
    dis2 = jnp.where(degt > 0.0, lax.rsqrt(degt), 0.0)
    ident = (jnp.sum(sc_blk, axis=1, keepdims=True) == 0.0).astype(jnp.float32)
    return dis1, dis2, ident


def _feat_body(x_ref, w1_ref, d1_ref, sc_ref, dt_ref, h_ref, hs1_ref, hs2_ref):
    h = jnp.dot(x_ref[...], w1_ref[...], preferred_element_type=jnp.float32)
    dis1, dis2, _ = _norms(d1_ref[...], sc_ref[...], dt_ref[...])
    z = jnp.zeros_like(h)
    h_ref[...] = h
    hs1_ref[...] = jnp.concatenate([h * dis1, z], axis=1)
    hs2_ref[...] = h * dis2


def _feat(xp, W1, deg1_t, self_t, degt_t, bm=1024):
    nm = NP // bm
    return pl.pallas_call(
        _feat_body,
        grid=(nm,),
        in_specs=[
            pl.BlockSpec((bm, 128), lambda i: (i, 0)),
            pl.BlockSpec((128, 64), lambda i: (0, 0)),
            pl.BlockSpec((bm, 32), lambda i: (i, 0)),
            pl.BlockSpec((bm, 32), lambda i: (i, 0)),
            pl.BlockSpec((bm, 1), lambda i: (i, 0)),
        ],
        out_specs=[
            pl.BlockSpec((bm, 64), lambda i: (i, 0)),
            pl.BlockSpec((bm, FW), lambda i: (i, 0)),
            pl.BlockSpec((bm, 64), lambda i: (i, 0)),
        ],
        out_shape=[
            jax.ShapeDtypeStruct((NP, 64), jnp.float32),
            jax.ShapeDtypeStruct((NP, FW), jnp.float32),
            jax.ShapeDtypeStruct((NP, 64), jnp.float32),
        ],
    )(xp, W1, deg1_t, self_t, degt_t)


def _layer1_body(a0_ref, a2_ref, h_ref, d1_ref, sc_ref, dt_ref,
                 b1_ref, w2_ref, g_ref, gs1_ref, gs2_ref):
    dis1, dis2, ident = _norms(d1_ref[...], sc_ref[...], dt_ref[...])
    b1 = b1_ref[0:1, :]
    h1 = dis1 * a0_ref[...] + b1
    h12 = dis2 * a2_ref[...] + b1
    h13 = ident * h_ref[...] + b1
    r1 = jax.nn.relu(jnp.concatenate([h1, h12, h13], axis=1))
    g = jnp.dot(r1, w2_ref[...], preferred_element_type=jnp.float32)
    z = jnp.zeros((g.shape[0], FW - 16), jnp.float32)
    g_ref[...] = g
    gs1_ref[...] = jnp.concatenate([g * dis1, z], axis=1)
    gs2_ref[...] = g * dis2


def _layer1(a0, a2, h, deg1_t, self_t, degt_t, b1r, W2, bm=1024):
    nm = NP // bm
    return pl.pallas_call(
        _layer1_body,
        grid=(nm,),
        in_specs=[
            pl.BlockSpec((bm, 64), lambda i: (i, 0)),
            pl.BlockSpec((bm, 64), lambda i: (i, 0)),
            pl.BlockSpec((bm, 64), lambda i: (i, 0)),
            pl.BlockSpec((bm, 32), lambda i: (i, 0)),
            pl.BlockSpec((bm, 32), lambda i: (i, 0)),
            pl.BlockSpec((bm, 1), lambda i: (i, 0)),
            pl.BlockSpec((1, 64), lambda i: (0, 0)),
            pl.BlockSpec((192, 16), lambda i: (0, 0)),
        ],
        out_specs=[
            pl.BlockSpec((bm, 16), lambda i: (i, 0)),
            pl.BlockSpec((bm, FW), lambda i: (i, 0)),
            pl.BlockSpec((bm, 16), lambda i: (i, 0)),
        ],
        out_shape=[
            jax.ShapeDtypeStruct((NP, 16), jnp.float32),
            jax.ShapeDtypeStruct((NP, FW), jnp.float32),
            jax.ShapeDtypeStruct((NP, 16), jnp.float32),
        ],
    )(a0, a2, h, deg1_t, self_t, degt_t, b1r, W2)


def _final_body(p0_ref, p2_ref, g_ref, d1_ref, sc_ref, dt_ref,
                b2_ref, wl_ref, bl_ref, o_ref):
    dis1, dis2, ident = _norms(d1_ref[...], sc_ref[...], dt_ref[...])
    b2 = b2_ref[0:1, :]
    h2 = dis1 * p0_ref[...] + b2
    h22 = dis2 * p2_ref[...] + b2
    h23 = ident * g_ref[...] + b2
    r2 = jnp.concatenate([h2, h22, h23], axis=1)
    fh = jnp.dot(r2, wl_ref[...], preferred_element_type=jnp.float32)
    fh = fh + bl_ref[0:1, :]
    m = jnp.max(fh, axis=1, keepdims=True)
    lse = m + jnp.log(jnp.sum(jnp.exp(fh - m), axis=1, keepdims=True))
    o_ref[...] = fh - lse


def _final(p0, p2, g, deg1_t, self_t, degt_t, b2r, Wlin, blinr, bm=1024):
    nm = NP // bm
    return pl.pallas_call(
        _final_body,
        grid=(nm,),
        in_specs=[
            pl.BlockSpec((bm, 16), lambda i: (i, 0)),
            pl.BlockSpec((bm, 16), lambda i: (i, 0)),
            pl.BlockSpec((bm, 16), lambda i: (i, 0)),
            pl.BlockSpec((bm, 32), lambda i: (i, 0)),
            pl.BlockSpec((bm, 32), lambda i: (i, 0)),
            pl.BlockSpec((bm, 1), lambda i: (i, 0)),
            pl.BlockSpec((1, 16), lambda i: (0, 0)),
            pl.BlockSpec((48, 16), lambda i: (0, 0)),
            pl.BlockSpec((1, 16), lambda i: (0, 0)),
        ],
        out_specs=pl.BlockSpec((bm, 16), lambda i: (i, 0)),
        out_shape=jax.ShapeDtypeStruct((NP, 16), jnp.float32),
    )(p0, p2, g, deg1_t, self_t, degt_t, b2r, Wlin, blinr)


def kernel(x, edge_index, W1, b1, W2, b2, Wlin, blin):
    xp = jnp.concatenate(
        [x, jnp.zeros((NP - N, x.shape[1]), x.dtype)], axis=0)

    a3, degp, selfp = _sc_edges(edge_index)
    abf = _to_bf16(a3)
    tmat, degt = _two_hop(abf)

    deg1_t = degp.reshape(32, NPH)[:, :NP].T
    self_t = selfp.reshape(32, NPH)[:, :NP].T
    degt_t = degt[0:1, :].T

    h, hs1, hs2 = _feat(xp, W1, deg1_t, self_t, degt_t)

    agg1 = _sc_prop(edge_index, hs1)
    a1full = jnp.concatenate([agg1[0, :HALF, :64], agg1[1, :HALF, :64]], 0)
    agg2 = _dense_prop(tmat, hs2)

    g, gs1, gs2 = _layer1(a1full, agg2, h,
                          deg1_t, self_t, degt_t,
                          b1.reshape(1, 64), W2)

    bgg1 = _sc_prop(edge_index, gs1)
    b1full = jnp.concatenate([bgg1[0, :HALF, :16], bgg1[1, :HALF, :16]], 0)
    bgg2 = _dense_prop(tmat, gs2)

    out = _final(b1full, bgg2, g,
                 deg1_t, self_t, degt_t,
                 b2.reshape(1, 16), Wlin, blin.reshape(1, 16))
    return out[:N]
